# Initial kernel scaffold; baseline (speedup 1.0000x reference)
import jax
import jax.numpy as jnp
from jax.experimental import pallas as pl


def kernel(src):
    raise NotImplementedError("write your pallas kernel here")


# speedup vs baseline: 68.2218x; 68.2218x over previous
import jax
import jax.numpy as jnp
from jax import lax
from jax.experimental import pallas as pl
from jax.experimental.pallas import tpu as pltpu
from jax.experimental.pallas import tpu_sc as plsc

NTOKEN = 256
MAX_PREFIX = 7
D_MODEL = 2048
SEQ_LEN = 2048
BATCH = 4

ROWS = SEQ_LEN * BATCH
NC, NS, L = 2, 16, 16
NW = NC * NS
ROWS_PER_W = ROWS // NW
GROUPS = ROWS_PER_W // L
PAD = 32
NSEG = MAX_PREFIX + 1
BUF = L * D_MODEL


def _body(src_hbm, out_hbm, src_v, buf0, buf1, sem0, sem1):
    wid = lax.axis_index("s") * NC + lax.axis_index("c")
    zeros16_i = jnp.zeros((L,), jnp.int32)
    zeros16_f = jnp.zeros((L,), jnp.float32)
    ones16_f = jnp.ones((L,), jnp.float32)
    lane = lax.iota(jnp.int32, L)
    lane_off = lane * D_MODEL

    src_v[pl.ds(0, L)] = zeros16_i
    src_v[pl.ds(L, L)] = zeros16_i
    pltpu.sync_copy(src_hbm, src_v.at[pl.ds(PAD, ROWS)])

    bufs = (buf0, buf1)
    sems = (sem0, sem1)

    @pl.loop(0, BUF // L)
    def _zero(c):
        buf0[pl.ds(c * L, L)] = zeros16_f
        buf1[pl.ds(c * L, L)] = zeros16_f

    row_base = wid * ROWS_PER_W

    def seg_cols_vals(r0):
        out = []
        for i in range(NSEG):
            cols = src_v[pl.ds(PAD + r0 - 4 * i, L)] + (i * NTOKEN)
            vals = jnp.where(r0 + lane >= 4 * i, ones16_f, zeros16_f)
            out.append((lane_off + cols, vals))
        return out

    copies = [None, None]
    for k in range(GROUPS):
        b = k % 2
        buf = bufs[b]
        r0 = row_base + k * L
        if k >= 2:
            copies[b].wait()
            r0_old = row_base + (k - 2) * L
            for idx, _ in seg_cols_vals(r0_old):
                plsc.store_scatter(buf, [idx], zeros16_f)
        for idx, vals in seg_cols_vals(r0):
            plsc.store_scatter(buf, [idx], vals)
        copies[b] = pltpu.async_copy(
            buf, out_hbm.at[pl.ds(r0 * D_MODEL, BUF)], sems[b]
        )
    copies[0].wait()
    copies[1].wait()


@jax.jit
def kernel(src):
    mesh = plsc.VectorSubcoreMesh(
        core_axis_name="c", subcore_axis_name="s", num_cores=NC, num_subcores=NS
    )
    k = pl.kernel(
        _body,
        out_type=jax.ShapeDtypeStruct((ROWS * D_MODEL,), jnp.float32),
        mesh=mesh,
        scratch_types=[
            pltpu.VMEM((PAD + ROWS,), jnp.int32),
            pltpu.VMEM((BUF,), jnp.float32),
            pltpu.VMEM((BUF,), jnp.float32),
            pltpu.SemaphoreType.DMA,
            pltpu.SemaphoreType.DMA,
        ],
        compiler_params=pltpu.CompilerParams(needs_layout_passes=False),
    )
    out = k(src.reshape(ROWS))
    return out.reshape(SEQ_LEN, BATCH, D_MODEL)
